# baseline (speedup 1.0000x reference)
import jax
import jax.numpy as jnp
from jax.experimental import pallas as pl
from jax.experimental.pallas import tpu as pltpu

_BROWS = 500_000
_BLOCK = 20_000


def _copy_body(src_ref, dst_ref):
    dst_ref[...] = src_ref[...]


def kernel(med_embeddings):
    n, d = med_embeddings.shape
    x8 = jax.lax.bitcast_convert_type(med_embeddings, jnp.int8)
    wide = x8.reshape(_BROWS, 128)
    out = pl.pallas_call(
        _copy_body,
        grid=(_BROWS // _BLOCK,),
        in_specs=[pl.BlockSpec((_BLOCK, 128), lambda i: (i, 0))],
        out_specs=pl.BlockSpec((_BLOCK, 128), lambda i: (i, 0)),
        out_shape=jax.ShapeDtypeStruct(wide.shape, wide.dtype),
    )(wide)
    return jax.lax.bitcast_convert_type(out.reshape(n, d, 4), jnp.float32)


# speedup vs baseline: 43.6589x; 43.6589x over previous
import jax
import jax.numpy as jnp
from jax.experimental import pallas as pl
from jax.experimental.pallas import tpu as pltpu

_ROWS = 1_000_000
_D = 16
_CH = 8_000
_NSTEPS = _ROWS // _CH
_NBUF = 6
_LAG = 3


def _copy_body(src, dst, *bufs_and_sems):
    bufs = bufs_and_sems[:_NBUF]
    sem_in, sem_out = bufs_and_sems[_NBUF], bufs_and_sems[_NBUF + 1]

    in_c = [None] * _NSTEPS
    out_c = [None] * _NSTEPS

    def issue_out(j):
        b = j % _NBUF
        in_c[j].wait()
        out_c[j] = pltpu.make_async_copy(
            bufs[b], dst.at[pl.ds(j * _CH, _CH), :], sem_out.at[b]
        )
        out_c[j].start()

    for i in range(_NSTEPS):
        b = i % _NBUF
        if i >= _NBUF:
            out_c[i - _NBUF].wait()
        in_c[i] = pltpu.make_async_copy(
            src.at[pl.ds(i * _CH, _CH), :], bufs[b], sem_in.at[b]
        )
        in_c[i].start()
        if i >= _LAG:
            issue_out(i - _LAG)
    for j in range(_NSTEPS - _LAG, _NSTEPS):
        issue_out(j)
    for j in range(_NSTEPS - _NBUF, _NSTEPS):
        out_c[j].wait()


def kernel(med_embeddings):
    return pl.pallas_call(
        _copy_body,
        in_specs=[pl.BlockSpec(memory_space=pltpu.MemorySpace.HBM)],
        out_specs=pl.BlockSpec(memory_space=pltpu.MemorySpace.HBM),
        out_shape=jax.ShapeDtypeStruct(med_embeddings.shape, med_embeddings.dtype),
        scratch_shapes=(
            [pltpu.VMEM((_CH, _D), jnp.float32) for _ in range(_NBUF)]
            + [pltpu.SemaphoreType.DMA((_NBUF,)), pltpu.SemaphoreType.DMA((_NBUF,))]
        ),
    )(med_embeddings)
